# baseline (speedup 1.0000x reference)
import numpy as np
import jax
import jax.numpy as jnp
from jax.experimental import pallas as pl
from jax.experimental.pallas import tpu as pltpu


def _interp_matrix(out_size, in_size, align_corners=False):
    if in_size == 1:
        return np.ones((out_size, 1), np.float32)
    if align_corners:
        src = np.arange(out_size, dtype=np.float64) * (in_size - 1) / max(out_size - 1, 1)
    else:
        src = (np.arange(out_size, dtype=np.float64) + 0.5) * (in_size / out_size) - 0.5
        src = np.clip(src, 0.0, in_size - 1.0)
    i0 = np.clip(np.floor(src).astype(np.int64), 0, in_size - 2)
    frac = (src - i0).astype(np.float32)
    m = np.zeros((out_size, in_size), np.float32)
    m[np.arange(out_size), i0] += 1.0 - frac
    m[np.arange(out_size), i0 + 1] += frac
    return m


def _band_weights(a, f):
    hh, hl = a.shape
    wm = np.zeros(hh, np.float32)
    w0 = np.zeros(hh, np.float32)
    wp = np.zeros(hh, np.float32)
    for i in range(hh):
        k = i // f
        if k - 1 >= 0:
            wm[i] = a[i, k - 1]
        w0[i] = a[i, k]
        if k + 1 < hl:
            wp[i] = a[i, k + 1]
    assert np.allclose(wm + w0 + wp, a.sum(axis=1)), "A support exceeds band"
    return wm, w0, wp


def _fused_kernel(x_ref, w_ref, bt_ref, wgt_ref, xh_ref, o_ref, yx_ref):
    cout, cin = w_ref.shape
    wl = bt_ref.shape[0]
    hl = x_ref.shape[2] // wl
    th, wh = xh_ref.shape[2], xh_ref.shape[3]
    fac = yx_ref.shape[1] // (hl + 2)

    t = pl.program_id(1)

    @pl.when(t == 0)
    def _prep():
        y = jnp.dot(w_ref[...].astype(jnp.float32),
                    x_ref[0].astype(jnp.float32),
                    preferred_element_type=jnp.float32)
        y3 = y.reshape(cout, hl, wl)
        yw3 = jax.lax.dot_general(
            y3, bt_ref[...], (((2,), (0,)), ((), ())),
            preferred_element_type=jnp.float32)
        yw4 = jnp.repeat(yw3, fac, axis=1)
        yx_ref[:, :fac] = yw4[:, :fac]
        yx_ref[:, fac:fac + hl * fac] = yw4
        yx_ref[:, fac + hl * fac:] = yw4[:, -fac:]

    base = t * th
    wm = wgt_ref[0][None]
    w0 = wgt_ref[1][None]
    wp = wgt_ref[2][None]
    up = (wm * yx_ref[:, pl.ds(base, th)]
          + w0 * yx_ref[:, pl.ds(base + fac, th)]
          + wp * yx_ref[:, pl.ds(base + 2 * fac, th)])

    o_ref[0, :cout] = up.astype(o_ref.dtype)
    o_ref[0, cout:] = xh_ref[0]


def _pick_row_tile(hh):
    if hh % 8 != 0:
        return hh
    best = 8
    for t in range(8, hh + 1, 8):
        if hh % t == 0 and hh // t >= 2 and t <= 32:
            best = t
    return best


def kernel(input_low, input_high, w):
    n, cin, hl, wl = input_low.shape
    nh, ch, hh, wh = input_high.shape
    cout = w.shape[0]
    ctot = cout + ch
    dtype = input_high.dtype
    isz = jnp.dtype(dtype).itemsize
    f = hh // hl
    assert f * hl == hh, "integer height scale factor expected"

    a = _interp_matrix(hh, hl)
    bt = jnp.asarray(_interp_matrix(wh, wl).T)
    wm, w0, wp = _band_weights(a, f)
    wgt = jnp.asarray(np.broadcast_to(
        np.stack([wm, w0, wp])[:, :, None], (3, hh, wh)).copy())

    th = _pick_row_tile(hh)
    n_t = hh // th

    blocks = (cin * hl * wl * isz + wl * wh * 4 + 3 * th * wh * 4
              + ch * th * wh * isz + ctot * th * wh * isz)
    scratch = 4 * cout * (hh + 2 * f) * wh + 4 * cout * (hl * wl + hl * wh + hh * wh)
    vmem_limit = int(min(63 << 20, max(32 << 20, 2 * 2 * blocks + scratch)))

    out = pl.pallas_call(
        _fused_kernel,
        out_shape=jax.ShapeDtypeStruct((n, ctot, hh, wh), dtype),
        grid=(n, n_t),
        in_specs=[
            pl.BlockSpec((1, cin, hl * wl), lambda i, t: (i, 0, 0)),
            pl.BlockSpec((cout, cin), lambda i, t: (0, 0)),
            pl.BlockSpec((wl, wh), lambda i, t: (0, 0)),
            pl.BlockSpec((3, th, wh), lambda i, t: (0, t, 0)),
            pl.BlockSpec((1, ch, th, wh), lambda i, t: (i, 0, t, 0)),
        ],
        out_specs=pl.BlockSpec((1, ctot, th, wh), lambda i, t: (i, 0, t, 0)),
        scratch_shapes=[pltpu.VMEM((cout, hh + 2 * f, wh), jnp.float32)],
        compiler_params=pltpu.CompilerParams(
            dimension_semantics=("parallel", "arbitrary"),
            vmem_limit_bytes=vmem_limit),
        cost_estimate=pl.CostEstimate(
            flops=int(2 * n * cout * (cin * hl * wl + hl * wl * wh)
                      + 6 * n * cout * hh * wh),
            transcendentals=0,
            bytes_accessed=int(isz * n * (cin * hl * wl + ch * hh * wh
                                          + ctot * hh * wh)
                               + 4 * (3 * hh * wh + wl * wh))),
    )(input_low.reshape(n, cin, hl * wl), w, bt, wgt, input_high)
    return out


# speedup vs baseline: 1.1486x; 1.1486x over previous
import numpy as np
import jax
import jax.numpy as jnp
from jax.experimental import pallas as pl
from jax.experimental.pallas import tpu as pltpu


def _interp_matrix(out_size, in_size, align_corners=False):
    if in_size == 1:
        return np.ones((out_size, 1), np.float32)
    if align_corners:
        src = np.arange(out_size, dtype=np.float64) * (in_size - 1) / max(out_size - 1, 1)
    else:
        src = (np.arange(out_size, dtype=np.float64) + 0.5) * (in_size / out_size) - 0.5
        src = np.clip(src, 0.0, in_size - 1.0)
    i0 = np.clip(np.floor(src).astype(np.int64), 0, in_size - 2)
    frac = (src - i0).astype(np.float32)
    m = np.zeros((out_size, in_size), np.float32)
    m[np.arange(out_size), i0] += 1.0 - frac
    m[np.arange(out_size), i0 + 1] += frac
    return m


def _fused_kernel(x_ref, w_ref, a_ref, bt_ref, xh_ref, o_ref, yt_ref, sem):
    cout = w_ref.shape[0]
    hl, wl = yt_ref.shape[0], bt_ref.shape[0]
    th, wh = a_ref.shape[0], bt_ref.shape[1]
    ch = o_ref.shape[1] - cout
    i = pl.program_id(0)
    t = pl.program_id(1)

    copy = pltpu.make_async_copy(
        xh_ref.at[i, :, pl.ds(t * th, th), :],
        o_ref.at[0, pl.ds(cout, ch)],
        sem)
    copy.start()

    @pl.when(pl.program_id(1) == 0)
    def _conv():
        y = jnp.dot(w_ref[...].astype(jnp.float32),
                    x_ref[0].astype(jnp.float32),
                    preferred_element_type=jnp.float32)
        yt = y.reshape(cout, hl, wl).transpose(1, 0, 2)
        yt_ref[...] = yt.reshape(hl, cout * wl)

    zt = jnp.dot(a_ref[...], yt_ref[...],
                 preferred_element_type=jnp.float32)
    zh = zt.reshape(th, cout, wl).transpose(1, 0, 2)
    up = jnp.dot(zh.reshape(cout * th, wl), bt_ref[...],
                 preferred_element_type=jnp.float32)

    o_ref[0, :cout] = up.reshape(cout, th, wh).astype(o_ref.dtype)
    copy.wait()


def _pick_row_tile(hh):
    if hh % 8 != 0:
        return hh
    best = 8
    for t in range(8, hh + 1, 8):
        if hh % t == 0 and hh // t >= 2 and t <= 64:
            best = t
    return best


def kernel(input_low, input_high, w):
    n, cin, hl, wl = input_low.shape
    nh, ch, hh, wh = input_high.shape
    cout = w.shape[0]
    ctot = cout + ch
    dtype = input_high.dtype
    isz = jnp.dtype(dtype).itemsize

    a = jnp.asarray(_interp_matrix(hh, hl))
    bt = jnp.asarray(_interp_matrix(wh, wl).T)

    th = _pick_row_tile(hh)
    n_t = hh // th

    blocks = (cin * hl * wl * isz + th * hl * 4 + wl * wh * 4
              + ch * th * wh * isz + ctot * th * wh * isz)
    scratch = 4 * hl * cout * wl + 4 * cout * (hl * wl + th * (hl + wl + 2 * wh))
    vmem_limit = int(min(120 << 20, max(32 << 20, 2 * (2 * blocks + scratch))))

    out = pl.pallas_call(
        _fused_kernel,
        out_shape=jax.ShapeDtypeStruct((n, ctot, hh, wh), dtype),
        grid=(n, n_t),
        in_specs=[
            pl.BlockSpec((1, cin, hl * wl), lambda i, t: (i, 0, 0)),
            pl.BlockSpec((cout, cin), lambda i, t: (0, 0)),
            pl.BlockSpec((th, hl), lambda i, t: (t, 0)),
            pl.BlockSpec((wl, wh), lambda i, t: (0, 0)),
            pl.BlockSpec(memory_space=pl.ANY),
        ],
        out_specs=pl.BlockSpec((1, ctot, th, wh), lambda i, t: (i, 0, t, 0)),
        scratch_shapes=[pltpu.VMEM((hl, cout * wl), jnp.float32),
                        pltpu.SemaphoreType.DMA],
        compiler_params=pltpu.CompilerParams(
            dimension_semantics=("parallel", "arbitrary"),
            vmem_limit_bytes=vmem_limit),
        cost_estimate=pl.CostEstimate(
            flops=int(2 * n * cout * (cin * hl * wl + hh * hl * wl + hh * wl * wh)),
            transcendentals=0,
            bytes_accessed=int(isz * n * (cin * hl * wl + ch * hh * wh
                                          + ctot * hh * wh)
                               + 4 * (hh * hl + wl * wh))),
    )(input_low.reshape(n, cin, hl * wl), w, a, bt, input_high)
    return out


# speedup vs baseline: 1.1528x; 1.0037x over previous
import numpy as np
import jax
import jax.numpy as jnp
from jax.experimental import pallas as pl
from jax.experimental.pallas import tpu as pltpu


def _interp_matrix(out_size, in_size, align_corners=False):
    if in_size == 1:
        return np.ones((out_size, 1), np.float32)
    if align_corners:
        src = np.arange(out_size, dtype=np.float64) * (in_size - 1) / max(out_size - 1, 1)
    else:
        src = (np.arange(out_size, dtype=np.float64) + 0.5) * (in_size / out_size) - 0.5
        src = np.clip(src, 0.0, in_size - 1.0)
    i0 = np.clip(np.floor(src).astype(np.int64), 0, in_size - 2)
    frac = (src - i0).astype(np.float32)
    m = np.zeros((out_size, in_size), np.float32)
    m[np.arange(out_size), i0] += 1.0 - frac
    m[np.arange(out_size), i0 + 1] += frac
    return m


def _fused_kernel(x_ref, w_ref, a_ref, bt_ref, xh_ref, o_ref, yt_ref):
    cout = w_ref.shape[0]
    hl, wl = yt_ref.shape[1], bt_ref.shape[0]
    th, wh = a_ref.shape[0], bt_ref.shape[1]
    i = pl.program_id(1)

    @pl.when(pl.program_id(0) == 0)
    def _conv():
        y = jnp.dot(w_ref[...].astype(jnp.float32),
                    x_ref[0].astype(jnp.float32),
                    preferred_element_type=jnp.float32)
        yt = y.reshape(cout, hl, wl).transpose(1, 0, 2)
        yt_ref[i] = yt.reshape(hl, cout * wl).astype(yt_ref.dtype)

    zt = jnp.dot(a_ref[...].astype(yt_ref.dtype), yt_ref[i],
                 preferred_element_type=jnp.float32)
    zh = zt.reshape(th, cout, wl).transpose(1, 0, 2)
    up = jnp.dot(zh.reshape(cout * th, wl), bt_ref[...],
                 preferred_element_type=jnp.float32)

    o_ref[0, :cout] = up.reshape(cout, th, wh).astype(o_ref.dtype)
    o_ref[0, cout:] = xh_ref[0]


def _pick_row_tile(hh):
    if hh % 8 != 0:
        return hh
    best = 8
    for t in range(8, hh + 1, 8):
        if hh % t == 0 and hh // t >= 2 and t <= 64:
            best = t
    return best


def kernel(input_low, input_high, w):
    n, cin, hl, wl = input_low.shape
    nh, ch, hh, wh = input_high.shape
    cout = w.shape[0]
    ctot = cout + ch
    dtype = input_high.dtype
    isz = jnp.dtype(dtype).itemsize

    a = jnp.asarray(_interp_matrix(hh, hl))
    bt = jnp.asarray(_interp_matrix(wh, wl).T)

    th = _pick_row_tile(hh)
    n_t = hh // th

    blocks = (cin * hl * wl * isz + th * hl * 4 + wl * wh * 4
              + ch * th * wh * isz + ctot * th * wh * isz)
    scratch = 4 * hl * cout * wl + 4 * cout * (hl * wl + th * (hl + wl + 2 * wh))
    vmem_limit = int(min(120 << 20, max(32 << 20, 2 * (2 * blocks + scratch))))

    out = pl.pallas_call(
        _fused_kernel,
        out_shape=jax.ShapeDtypeStruct((n, ctot, hh, wh), dtype),
        grid=(n_t, n),
        in_specs=[
            pl.BlockSpec((1, cin, hl * wl), lambda t, i: (i, 0, 0)),
            pl.BlockSpec((cout, cin), lambda t, i: (0, 0)),
            pl.BlockSpec((th, hl), lambda t, i: (t, 0)),
            pl.BlockSpec((wl, wh), lambda t, i: (0, 0)),
            pl.BlockSpec((1, ch, th, wh), lambda t, i: (i, 0, t, 0)),
        ],
        out_specs=pl.BlockSpec((1, ctot, th, wh), lambda t, i: (i, 0, t, 0)),
        scratch_shapes=[pltpu.VMEM((n, hl, cout * wl), jnp.bfloat16)],
        compiler_params=pltpu.CompilerParams(
            dimension_semantics=("arbitrary", "arbitrary"),
            vmem_limit_bytes=vmem_limit),
        cost_estimate=pl.CostEstimate(
            flops=int(2 * n * cout * (cin * hl * wl + hh * hl * wl + hh * wl * wh)),
            transcendentals=0,
            bytes_accessed=int(isz * n * (cin * hl * wl + ch * hh * wh
                                          + ctot * hh * wh)
                               + 4 * (hh * hl + wl * wh))),
    )(input_low.reshape(n, cin, hl * wl), w, a, bt, input_high)
    return out


# speedup vs baseline: 1.1728x; 1.0174x over previous
import numpy as np
import jax
import jax.numpy as jnp
from jax.experimental import pallas as pl
from jax.experimental.pallas import tpu as pltpu


def _interp_matrix(out_size, in_size, align_corners=False):
    if in_size == 1:
        return np.ones((out_size, 1), np.float32)
    if align_corners:
        src = np.arange(out_size, dtype=np.float64) * (in_size - 1) / max(out_size - 1, 1)
    else:
        src = (np.arange(out_size, dtype=np.float64) + 0.5) * (in_size / out_size) - 0.5
        src = np.clip(src, 0.0, in_size - 1.0)
    i0 = np.clip(np.floor(src).astype(np.int64), 0, in_size - 2)
    frac = (src - i0).astype(np.float32)
    m = np.zeros((out_size, in_size), np.float32)
    m[np.arange(out_size), i0] += 1.0 - frac
    m[np.arange(out_size), i0 + 1] += frac
    return m


def _fused_kernel(x_ref, w_ref, a_ref, bt_ref, xh_ref, o_ref, yt_ref):
    cout = w_ref.shape[0]
    hl, wl = yt_ref.shape[0], bt_ref.shape[0]
    th, wh = xh_ref.shape[2], bt_ref.shape[1]
    t = pl.program_id(1)

    @pl.when(t == 0)
    def _conv():
        y = jnp.dot(w_ref[...].astype(jnp.float32),
                    x_ref[0].astype(jnp.float32),
                    preferred_element_type=jnp.float32)
        yt = y.reshape(cout, hl, wl).transpose(1, 0, 2)
        yt_ref[...] = yt.reshape(hl, cout * wl)

    zt = jnp.dot(a_ref[pl.ds(t * th, th)], yt_ref[...],
                 preferred_element_type=jnp.float32)
    zh = zt.reshape(th, cout, wl).transpose(1, 0, 2)
    up = jnp.dot(zh.reshape(cout * th, wl), bt_ref[...],
                 preferred_element_type=jnp.float32)

    o_ref[0, :cout] = up.reshape(cout, th, wh).astype(o_ref.dtype)
    o_ref[0, cout:] = xh_ref[0]


def _pick_row_tile(hh):
    if hh % 8 != 0:
        return hh
    best = 8
    for t in range(8, hh + 1, 8):
        if hh % t == 0 and hh // t >= 2 and t <= 64:
            best = t
    return best


def kernel(input_low, input_high, w):
    n, cin, hl, wl = input_low.shape
    nh, ch, hh, wh = input_high.shape
    cout = w.shape[0]
    ctot = cout + ch
    dtype = input_high.dtype
    isz = jnp.dtype(dtype).itemsize

    a = jnp.asarray(_interp_matrix(hh, hl))
    bt = jnp.asarray(_interp_matrix(wh, wl).T)

    th = _pick_row_tile(hh)
    n_t = hh // th

    blocks = (cin * hl * wl * isz + th * hl * 4 + wl * wh * 4
              + ch * th * wh * isz + ctot * th * wh * isz)
    scratch = 4 * hl * cout * wl + 4 * cout * (hl * wl + th * (hl + wl + 2 * wh))
    vmem_limit = int(min(120 << 20, max(32 << 20, 2 * (2 * blocks + scratch))))

    out = pl.pallas_call(
        _fused_kernel,
        out_shape=jax.ShapeDtypeStruct((n, ctot, hh, wh), dtype),
        grid=(n, n_t),
        in_specs=[
            pl.BlockSpec((1, cin, hl * wl), lambda i, t: (i, 0, 0)),
            pl.BlockSpec((cout, cin), lambda i, t: (0, 0)),
            pl.BlockSpec((hh, hl), lambda i, t: (0, 0)),
            pl.BlockSpec((wl, wh), lambda i, t: (0, 0)),
            pl.BlockSpec((1, ch, th, wh), lambda i, t: (i, 0, t, 0)),
        ],
        out_specs=pl.BlockSpec((1, ctot, th, wh), lambda i, t: (i, 0, t, 0)),
        scratch_shapes=[pltpu.VMEM((hl, cout * wl), jnp.float32)],
        compiler_params=pltpu.CompilerParams(
            dimension_semantics=("parallel", "arbitrary"),
            vmem_limit_bytes=vmem_limit),
        cost_estimate=pl.CostEstimate(
            flops=int(2 * n * cout * (cin * hl * wl + hh * hl * wl + hh * wl * wh)),
            transcendentals=0,
            bytes_accessed=int(isz * n * (cin * hl * wl + ch * hh * wh
                                          + ctot * hh * wh)
                               + 4 * (hh * hl + wl * wh))),
    )(input_low.reshape(n, cin, hl * wl), w, a, bt, input_high)
    return out
